# Initial kernel scaffold; baseline (speedup 1.0000x reference)
import jax
import jax.numpy as jnp
from jax.experimental import pallas as pl


def kernel(x_user, x_movie, edge_index_rates, edge_index_rev, edge_weight_rates, edge_weight_rev, W1_um_l, b1_um, W1_um_r, W1_mu_l, b1_mu, W1_mu_r, W2_um_l, b2_um, W2_um_r, W2_mu_l, b2_mu, W2_mu_r):
    raise NotImplementedError("write your pallas kernel here")


# speedup vs baseline: 6.2001x; 6.2001x over previous
import jax
import jax.numpy as jnp
from jax import lax
from jax.experimental import pallas as pl
from jax.experimental.pallas import tpu as pltpu
from jax.experimental.pallas import tpu_sc as plsc

D = 128
LANES = 16
SUB = 128
KSUB = 2
NTILES = 16
CNT_W = 128


def _ceil_to(x, m):
    return (x + m - 1) // m * m


def _mesh():
    return plsc.VectorSubcoreMesh(core_axis_name="c", subcore_axis_name="s")


def _sc_segsum(n_pad, e_pad):
    rows_per_tile = n_pad // NTILES
    nzc = rows_per_tile // SUB
    iters = e_pad // (NTILES * SUB * KSUB)
    erows_per_tile = e_pad // (NTILES * SUB)

    out_type = [
        jax.ShapeDtypeStruct((n_pad, D), jnp.float32),
        jax.ShapeDtypeStruct((n_pad, D), jnp.float32),
    ]
    scratch = [
        pltpu.VMEM_SHARED((n_pad, D), jnp.float32),
        pltpu.VMEM((KSUB * SUB, D), jnp.float32),
        pltpu.VMEM((KSUB, SUB), jnp.int32),
        pltpu.VMEM((KSUB, SUB), jnp.int32),
        pltpu.SemaphoreType.DMA,
    ]

    def kern(us2d, ms2d, ud2d, md2d, tab_u, tab_m, out_m, out_u,
             acc, rows_v, sidx_v, didx_v, gsem):
        tid = lax.axis_index("s")
        cid = lax.axis_index("c")

        def zrow(i, _):
            r = i // (D // LANES)
            c = lax.rem(i, D // LANES)
            rows_v[r, pl.ds(c * LANES, LANES)] = jnp.zeros((LANES,), jnp.float32)
            return 0
        lax.fori_loop(0, SUB * (D // LANES), zrow, 0)
        for q in range(nzc):
            pltpu.sync_copy(rows_v.at[pl.ds(0, SUB)],
                            acc.at[pl.ds(tid * rows_per_tile + q * SUB, SUB)])
        plsc.subcore_barrier()

        def edge_loop(tab, s2d, d2d):
            def outer(g, _):
                base = tid * erows_per_tile + g * KSUB
                pltpu.sync_copy(s2d.at[pl.ds(base, KSUB)], sidx_v)
                pltpu.sync_copy(d2d.at[pl.ds(base, KSUB)], didx_v)
                cps = [
                    pltpu.async_copy(tab.at[sidx_v.at[j]],
                                     rows_v.at[pl.ds(j * SUB, SUB)], gsem)
                    for j in range(KSUB)
                ]
                for cp in cps:
                    cp.wait()
                for j in range(KSUB):
                    pltpu.sync_copy(rows_v.at[pl.ds(j * SUB, SUB)],
                                    acc.at[didx_v.at[j]], add=True)
                return 0
            lax.fori_loop(0, iters, outer, 0)

        @pl.when(cid == 0)
        def _():
            edge_loop(tab_u, us2d, md2d)

        @pl.when(cid == 1)
        def _():
            edge_loop(tab_m, ms2d, ud2d)

        plsc.subcore_barrier()

        def readout(out_ref):
            for q in range(nzc):
                r0 = tid * rows_per_tile + q * SUB
                pltpu.sync_copy(acc.at[pl.ds(r0, SUB)], out_ref.at[pl.ds(r0, SUB)])

        @pl.when(cid == 0)
        def _():
            readout(out_m)

        @pl.when(cid == 1)
        def _():
            readout(out_u)

    return pl.kernel(kern, out_type=out_type, mesh=_mesh(),
                     scratch_types=scratch)


def _sc_counts(n_pad, e_pad):
    rows_per_tile = n_pad // NTILES
    nzc = rows_per_tile // SUB
    iters = e_pad // (NTILES * SUB * KSUB)
    erows_per_tile = e_pad // (NTILES * SUB)

    out_type = [
        jax.ShapeDtypeStruct((n_pad, CNT_W), jnp.float32),
        jax.ShapeDtypeStruct((n_pad, CNT_W), jnp.float32),
    ]
    scratch = [
        pltpu.VMEM_SHARED((n_pad, CNT_W), jnp.float32),
        pltpu.VMEM((SUB, CNT_W), jnp.float32),
        pltpu.VMEM((SUB, CNT_W), jnp.float32),
        pltpu.VMEM((KSUB, SUB), jnp.int32),
    ]

    def kern(ud2d, md2d, cnt_m, cnt_u, cacc, ones_v, zc_v, didx_v):
        tid = lax.axis_index("s")
        cid = lax.axis_index("c")

        def frow(i, _):
            r = i // (CNT_W // LANES)
            c = lax.rem(i, CNT_W // LANES)
            zc_v[r, pl.ds(c * LANES, LANES)] = jnp.zeros((LANES,), jnp.float32)
            ones_v[r, pl.ds(c * LANES, LANES)] = jnp.ones((LANES,), jnp.float32)
            return 0
        lax.fori_loop(0, SUB * (CNT_W // LANES), frow, 0)
        for q in range(nzc):
            pltpu.sync_copy(zc_v,
                            cacc.at[pl.ds(tid * rows_per_tile + q * SUB, SUB)])
        plsc.subcore_barrier()

        def cnt_loop(d2d):
            def outer(g, _):
                base = tid * erows_per_tile + g * KSUB
                pltpu.sync_copy(d2d.at[pl.ds(base, KSUB)], didx_v)
                for j in range(KSUB):
                    pltpu.sync_copy(ones_v, cacc.at[didx_v.at[j]], add=True)
                return 0
            lax.fori_loop(0, iters, outer, 0)

        @pl.when(cid == 0)
        def _():
            cnt_loop(md2d)

        @pl.when(cid == 1)
        def _():
            cnt_loop(ud2d)

        plsc.subcore_barrier()

        def readout(out_ref):
            for q in range(nzc):
                r0 = tid * rows_per_tile + q * SUB
                pltpu.sync_copy(cacc.at[pl.ds(r0, SUB)], out_ref.at[pl.ds(r0, SUB)])

        @pl.when(cid == 0)
        def _():
            readout(cnt_m)

        @pl.when(cid == 1)
        def _():
            readout(cnt_u)

    return pl.kernel(kern, out_type=out_type, mesh=_mesh(),
                     scratch_types=scratch)


def _tc_dense(s_m, cnt_m, x_m, w_l_m, b_m, w_r_m,
              s_u, cnt_u, x_u, w_l_u, b_u, w_r_u, residual):
    n = x_m.shape[0]

    def kern(sm, cm, xm, wlm, bm, wrm, su, cu, xu, wlu, bu, wru, om, ou):
        def one(s_ref, c_ref, x_ref, wl_ref, b_ref, wr_ref, o_ref):
            rc = 1.0 / jnp.clip(c_ref[...][:, 0:1], 1.0, None)
            mean = s_ref[...] * rc
            y = (jnp.dot(mean, wl_ref[...], preferred_element_type=jnp.float32)
                 + b_ref[...]
                 + jnp.dot(x_ref[...], wr_ref[...],
                           preferred_element_type=jnp.float32))
            if residual:
                y = x_ref[...] + jnp.maximum(y, 0.0)
            o_ref[...] = y
        one(sm, cm, xm, wlm, bm, wrm, om)
        one(su, cu, xu, wlu, bu, wru, ou)

    out = pl.pallas_call(
        kern,
        out_shape=[jax.ShapeDtypeStruct((n, D), jnp.float32),
                   jax.ShapeDtypeStruct((n, D), jnp.float32)],
    )(s_m, cnt_m, x_m, w_l_m, b_m, w_r_m, s_u, cnt_u, x_u, w_l_u, b_u, w_r_u)
    return out[0], out[1]


def kernel(x_user, x_movie, edge_index_rates, edge_index_rev,
           edge_weight_rates, edge_weight_rev,
           W1_um_l, b1_um, W1_um_r, W1_mu_l, b1_mu, W1_mu_r,
           W2_um_l, b2_um, W2_um_r, W2_mu_l, b2_mu, W2_mu_r):
    nu, d = x_user.shape
    nm = x_movie.shape[0]
    e = edge_index_rates.shape[1]
    assert d == D and nu == nm

    n_pad = _ceil_to(nu, NTILES * SUB)
    e_pad = _ceil_to(e, NTILES * SUB * KSUB)

    u_idx = edge_index_rates[0].astype(jnp.int32)
    m_idx = edge_index_rates[1].astype(jnp.int32)
    pad = e_pad - e
    if pad:
        fill = jnp.arange(pad, dtype=jnp.int32)
        dummy = nu + fill % (n_pad - nu)
        u_s = jnp.concatenate([u_idx, fill % nu])
        m_s = jnp.concatenate([m_idx, fill % nm])
        u_d = jnp.concatenate([u_idx, dummy])
        m_d = jnp.concatenate([m_idx, dummy])
    else:
        u_s = u_d = u_idx
        m_s = m_d = m_idx

    u_s = u_s.reshape(-1, SUB)
    m_s = m_s.reshape(-1, SUB)
    u_d = u_d.reshape(-1, SUB)
    m_d = m_d.reshape(-1, SUB)

    seg = _sc_segsum(n_pad, e_pad)
    c_m, c_u = _sc_counts(n_pad, e_pad)(u_d, m_d)

    s_m, s_u = seg(u_s, m_s, u_d, m_d, x_user, x_movie)
    res_movie, res_user = _tc_dense(
        s_m[:nm], c_m[:nm], x_movie, W1_um_l, b1_um.reshape(1, D), W1_um_r,
        s_u[:nu], c_u[:nu], x_user, W1_mu_l, b1_mu.reshape(1, D), W1_mu_r,
        residual=True)

    s2_m, s2_u = seg(u_s, m_s, u_d, m_d, res_user, res_movie)
    m2, u2 = _tc_dense(
        s2_m[:nm], c_m[:nm], res_movie, W2_um_l, b2_um.reshape(1, D), W2_um_r,
        s2_u[:nu], c_u[:nu], res_user, W2_mu_l, b2_mu.reshape(1, D), W2_mu_r,
        residual=False)

    return (u2, m2)


# speedup vs baseline: 7.9569x; 1.2834x over previous
import jax
import jax.numpy as jnp
from jax import lax
from jax.experimental import pallas as pl
from jax.experimental.pallas import tpu as pltpu
from jax.experimental.pallas import tpu_sc as plsc

D = 128
LANES = 16
SUB = 128
KSUB = 2
NTILES = 16
CNT_W = 128


def _ceil_to(x, m):
    return (x + m - 1) // m * m


def _mesh():
    return plsc.VectorSubcoreMesh(core_axis_name="c", subcore_axis_name="s")


GRP = 16


def _sc_segsum(n_pad, e_pad):
    rows_per_tile = n_pad // NTILES
    nzc = rows_per_tile // SUB
    erows_per_tile = e_pad // (NTILES * SUB)
    npairs = erows_per_tile // 2
    ngrp = erows_per_tile // GRP

    out_type = [
        jax.ShapeDtypeStruct((n_pad, D), jnp.float32),
        jax.ShapeDtypeStruct((n_pad, D), jnp.float32),
    ]
    scratch = [
        pltpu.VMEM_SHARED((n_pad, D), jnp.float32),
        pltpu.VMEM((SUB, D), jnp.float32),
        pltpu.VMEM((SUB, D), jnp.float32),
        pltpu.VMEM((GRP, SUB), jnp.int32),
        pltpu.VMEM((GRP, SUB), jnp.int32),
        pltpu.SemaphoreType.DMA,
        pltpu.SemaphoreType.DMA,
    ]

    def kern(us2d, ms2d, ud2d, md2d, tab_u, tab_m, out_m, out_u,
             acc, rows_a, rows_b, sidx_v, didx_v, sem_a, sem_b):
        tid = lax.axis_index("s")
        cid = lax.axis_index("c")
        ebase = tid * erows_per_tile

        def zrow(i, _):
            r = i // (D // LANES)
            c = lax.rem(i, D // LANES)
            rows_a[r, pl.ds(c * LANES, LANES)] = jnp.zeros((LANES,), jnp.float32)
            return 0
        lax.fori_loop(0, SUB * (D // LANES), zrow, 0)
        for q in range(nzc):
            pltpu.sync_copy(rows_a,
                            acc.at[pl.ds(tid * rows_per_tile + q * SUB, SUB)])
        plsc.subcore_barrier()

        def edge_loop(tab, s2d, d2d):
            def load_grp(grp):
                pltpu.sync_copy(s2d.at[pl.ds(ebase + grp * GRP, GRP)], sidx_v)
                pltpu.sync_copy(d2d.at[pl.ds(ebase + grp * GRP, GRP)], didx_v)

            def gather(r, buf, sem):
                return pltpu.async_copy(tab.at[sidx_v.at[r]], buf, sem)

            def drain(buf, sem):
                pltpu.make_async_copy(tab.at[pl.ds(0, SUB)], buf, sem).wait()

            def scatter(r, buf):
                pltpu.sync_copy(buf, acc.at[didx_v.at[r]], add=True)

            load_grp(0)
            gather(0, rows_a, sem_a)

            def pair(t, _):
                r0 = lax.rem(2 * t, GRP)
                drain(rows_a, sem_a)
                gather(r0 + 1, rows_b, sem_b)
                scatter(r0, rows_a)
                drain(rows_b, sem_b)

                at_grp_end = lax.rem(t + 1, GRP // 2) == 0

                @pl.when(jnp.logical_not(at_grp_end))
                def _():
                    gather(r0 + 2, rows_a, sem_a)
                    scatter(r0 + 1, rows_b)

                @pl.when(at_grp_end)
                def _():
                    scatter(r0 + 1, rows_b)

                    @pl.when(t + 1 < npairs)
                    def _():
                        load_grp((2 * t + 2) // GRP)
                        gather(0, rows_a, sem_a)

                return 0
            lax.fori_loop(0, npairs, pair, 0)

        @pl.when(cid == 0)
        def _():
            edge_loop(tab_u, us2d, md2d)

        @pl.when(cid == 1)
        def _():
            edge_loop(tab_m, ms2d, ud2d)

        plsc.subcore_barrier()

        def readout(out_ref):
            for q in range(nzc):
                r0 = tid * rows_per_tile + q * SUB
                pltpu.sync_copy(acc.at[pl.ds(r0, SUB)], out_ref.at[pl.ds(r0, SUB)])

        @pl.when(cid == 0)
        def _():
            readout(out_m)

        @pl.when(cid == 1)
        def _():
            readout(out_u)

    return pl.kernel(kern, out_type=out_type, mesh=_mesh(),
                     scratch_types=scratch)


def _sc_counts(n_pad, e_pad):
    rows_per_tile = n_pad // NTILES
    nzc = rows_per_tile // SUB
    iters = e_pad // (NTILES * SUB * KSUB)
    erows_per_tile = e_pad // (NTILES * SUB)

    out_type = [
        jax.ShapeDtypeStruct((n_pad, CNT_W), jnp.float32),
        jax.ShapeDtypeStruct((n_pad, CNT_W), jnp.float32),
    ]
    scratch = [
        pltpu.VMEM_SHARED((n_pad, CNT_W), jnp.float32),
        pltpu.VMEM((SUB, CNT_W), jnp.float32),
        pltpu.VMEM((SUB, CNT_W), jnp.float32),
        pltpu.VMEM((KSUB, SUB), jnp.int32),
    ]

    def kern(ud2d, md2d, cnt_m, cnt_u, cacc, ones_v, zc_v, didx_v):
        tid = lax.axis_index("s")
        cid = lax.axis_index("c")

        def frow(i, _):
            r = i // (CNT_W // LANES)
            c = lax.rem(i, CNT_W // LANES)
            zc_v[r, pl.ds(c * LANES, LANES)] = jnp.zeros((LANES,), jnp.float32)
            ones_v[r, pl.ds(c * LANES, LANES)] = jnp.ones((LANES,), jnp.float32)
            return 0
        lax.fori_loop(0, SUB * (CNT_W // LANES), frow, 0)
        for q in range(nzc):
            pltpu.sync_copy(zc_v,
                            cacc.at[pl.ds(tid * rows_per_tile + q * SUB, SUB)])
        plsc.subcore_barrier()

        def cnt_loop(d2d):
            def outer(g, _):
                base = tid * erows_per_tile + g * KSUB
                pltpu.sync_copy(d2d.at[pl.ds(base, KSUB)], didx_v)
                for j in range(KSUB):
                    pltpu.sync_copy(ones_v, cacc.at[didx_v.at[j]], add=True)
                return 0
            lax.fori_loop(0, iters, outer, 0)

        @pl.when(cid == 0)
        def _():
            cnt_loop(md2d)

        @pl.when(cid == 1)
        def _():
            cnt_loop(ud2d)

        plsc.subcore_barrier()

        def readout(out_ref):
            for q in range(nzc):
                r0 = tid * rows_per_tile + q * SUB
                pltpu.sync_copy(cacc.at[pl.ds(r0, SUB)], out_ref.at[pl.ds(r0, SUB)])

        @pl.when(cid == 0)
        def _():
            readout(cnt_m)

        @pl.when(cid == 1)
        def _():
            readout(cnt_u)

    return pl.kernel(kern, out_type=out_type, mesh=_mesh(),
                     scratch_types=scratch)


def _tc_dense(s_m, cnt_m, x_m, w_l_m, b_m, w_r_m,
              s_u, cnt_u, x_u, w_l_u, b_u, w_r_u, residual):
    n = x_m.shape[0]

    def kern(sm, cm, xm, wlm, bm, wrm, su, cu, xu, wlu, bu, wru, om, ou):
        def one(s_ref, c_ref, x_ref, wl_ref, b_ref, wr_ref, o_ref):
            rc = 1.0 / jnp.clip(c_ref[...][:n, 0:1], 1.0, None)
            mean = s_ref[...][:n] * rc
            y = (jnp.dot(mean, wl_ref[...], preferred_element_type=jnp.float32)
                 + b_ref[...]
                 + jnp.dot(x_ref[...], wr_ref[...],
                           preferred_element_type=jnp.float32))
            if residual:
                y = x_ref[...] + jnp.maximum(y, 0.0)
            o_ref[...] = y
        one(sm, cm, xm, wlm, bm, wrm, om)
        one(su, cu, xu, wlu, bu, wru, ou)

    out = pl.pallas_call(
        kern,
        out_shape=[jax.ShapeDtypeStruct((n, D), jnp.float32),
                   jax.ShapeDtypeStruct((n, D), jnp.float32)],
    )(s_m, cnt_m, x_m, w_l_m, b_m, w_r_m, s_u, cnt_u, x_u, w_l_u, b_u, w_r_u)
    return out[0], out[1]


def kernel(x_user, x_movie, edge_index_rates, edge_index_rev,
           edge_weight_rates, edge_weight_rev,
           W1_um_l, b1_um, W1_um_r, W1_mu_l, b1_mu, W1_mu_r,
           W2_um_l, b2_um, W2_um_r, W2_mu_l, b2_mu, W2_mu_r):
    nu, d = x_user.shape
    nm = x_movie.shape[0]
    e = edge_index_rates.shape[1]
    assert d == D and nu == nm

    n_pad = _ceil_to(nu, NTILES * SUB)
    e_pad = _ceil_to(e, NTILES * SUB * GRP)

    u_idx = edge_index_rates[0].astype(jnp.int32)
    m_idx = edge_index_rates[1].astype(jnp.int32)
    pad = e_pad - e
    if pad:
        fill = jnp.arange(pad, dtype=jnp.int32)
        dummy = nu + fill % (n_pad - nu)
        u_s = jnp.concatenate([u_idx, fill % nu])
        m_s = jnp.concatenate([m_idx, fill % nm])
        u_d = jnp.concatenate([u_idx, dummy])
        m_d = jnp.concatenate([m_idx, dummy])
    else:
        u_s = u_d = u_idx
        m_s = m_d = m_idx

    u_s = u_s.reshape(-1, SUB)
    m_s = m_s.reshape(-1, SUB)
    u_d = u_d.reshape(-1, SUB)
    m_d = m_d.reshape(-1, SUB)

    seg = _sc_segsum(n_pad, e_pad)
    c_m, c_u = _sc_counts(n_pad, e_pad)(u_d, m_d)

    s_m, s_u = seg(u_s, m_s, u_d, m_d, x_user, x_movie)
    res_movie, res_user = _tc_dense(
        s_m, c_m, x_movie, W1_um_l, b1_um.reshape(1, D), W1_um_r,
        s_u, c_u, x_user, W1_mu_l, b1_mu.reshape(1, D), W1_mu_r,
        residual=True)

    s2_m, s2_u = seg(u_s, m_s, u_d, m_d, res_user, res_movie)
    m2, u2 = _tc_dense(
        s2_m, c_m, res_movie, W2_um_l, b2_um.reshape(1, D), W2_um_r,
        s2_u, c_u, res_user, W2_mu_l, b2_mu.reshape(1, D), W2_mu_r,
        residual=False)

    return (u2, m2)


# speedup vs baseline: 8.2454x; 1.0363x over previous
import jax
import jax.numpy as jnp
from jax import lax
from jax.experimental import pallas as pl
from jax.experimental.pallas import tpu as pltpu
from jax.experimental.pallas import tpu_sc as plsc

D = 128
LANES = 16
SUB = 128
NTILES = 16


def _ceil_to(x, m):
    return (x + m - 1) // m * m


def _mesh():
    return plsc.VectorSubcoreMesh(core_axis_name="c", subcore_axis_name="s")


GRP = 16


def _sc_segsum(n_pad, e_pad):
    rows_per_tile = n_pad // NTILES
    nzc = rows_per_tile // SUB
    erows_per_tile = e_pad // (NTILES * SUB)
    npairs = erows_per_tile // 2
    ngrp = erows_per_tile // GRP

    out_type = [
        jax.ShapeDtypeStruct((n_pad, D), jnp.float32),
        jax.ShapeDtypeStruct((n_pad, D), jnp.float32),
    ]
    scratch = [
        pltpu.VMEM_SHARED((n_pad, D), jnp.float32),
        pltpu.VMEM((SUB, D), jnp.float32),
        pltpu.VMEM((SUB, D), jnp.float32),
        pltpu.VMEM((GRP, SUB), jnp.int32),
        pltpu.VMEM((GRP, SUB), jnp.int32),
        pltpu.SemaphoreType.DMA,
        pltpu.SemaphoreType.DMA,
        pltpu.SemaphoreType.DMA,
    ]

    def kern(us2d, ms2d, ud2d, md2d, tab_u, tab_m, out_m, out_u,
             acc, rows_a, rows_b, sidx_v, didx_v, sem_a, sem_b, ssem):
        tid = lax.axis_index("s")
        cid = lax.axis_index("c")
        ebase = tid * erows_per_tile

        def zrow(i, _):
            r = i // (D // LANES)
            c = lax.rem(i, D // LANES)
            rows_a[r, pl.ds(c * LANES, LANES)] = jnp.zeros((LANES,), jnp.float32)
            return 0
        lax.fori_loop(0, SUB * (D // LANES), zrow, 0)
        for q in range(nzc):
            pltpu.sync_copy(rows_a,
                            acc.at[pl.ds(tid * rows_per_tile + q * SUB, SUB)])

        plsc.subcore_barrier()

        def edge_loop(tab, s2d, d2d):
            def load_grp(grp):
                pltpu.sync_copy(s2d.at[pl.ds(ebase + grp * GRP, GRP)], sidx_v)
                pltpu.sync_copy(d2d.at[pl.ds(ebase + grp * GRP, GRP)], didx_v)

            def gather(r, buf, sem):
                return pltpu.async_copy(tab.at[sidx_v.at[r]], buf, sem)

            def drain_g(buf, sem):
                pltpu.make_async_copy(tab.at[pl.ds(0, SUB)], buf, sem).wait()

            def scatter(r, buf):
                pltpu.async_copy(buf, acc.at[didx_v.at[r]], ssem, add=True)

            def drain_s(buf):
                pltpu.make_async_copy(buf, acc.at[pl.ds(0, SUB)], ssem).wait()

            load_grp(0)
            gather(0, rows_a, sem_a)

            def pair(t, _):
                r0 = lax.rem(2 * t, GRP)
                drain_g(rows_a, sem_a)
                gather(r0 + 1, rows_b, sem_b)
                scatter(r0, rows_a)
                drain_g(rows_b, sem_b)
                drain_s(rows_a)

                at_grp_end = lax.rem(t + 1, GRP // 2) == 0

                @pl.when(jnp.logical_not(at_grp_end))
                def _():
                    gather(r0 + 2, rows_a, sem_a)
                    scatter(r0 + 1, rows_b)
                    drain_s(rows_b)

                @pl.when(at_grp_end)
                def _():
                    scatter(r0 + 1, rows_b)
                    drain_s(rows_b)

                    @pl.when(t + 1 < npairs)
                    def _():
                        load_grp((2 * t + 2) // GRP)
                        gather(0, rows_a, sem_a)

                return 0
            lax.fori_loop(0, npairs, pair, 0)

        @pl.when(cid == 0)
        def _():
            edge_loop(tab_u, us2d, md2d)

        @pl.when(cid == 1)
        def _():
            edge_loop(tab_m, ms2d, ud2d)

        plsc.subcore_barrier()

        def readout(out_ref):
            for q in range(nzc):
                r0 = tid * rows_per_tile + q * SUB
                pltpu.sync_copy(acc.at[pl.ds(r0, SUB)], out_ref.at[pl.ds(r0, SUB)])

        @pl.when(cid == 0)
        def _():
            readout(out_m)

        @pl.when(cid == 1)
        def _():
            readout(out_u)

    return pl.kernel(kern, out_type=out_type, mesh=_mesh(),
                     scratch_types=scratch)


CNT_W = 128


def _sc_counts(n_pad, e_pad):
    rows_per_tile = n_pad // NTILES
    nzc = rows_per_tile // SUB
    erows_per_tile = e_pad // (NTILES * SUB)

    out_type = [
        jax.ShapeDtypeStruct((n_pad, CNT_W), jnp.float32),
        jax.ShapeDtypeStruct((n_pad, CNT_W), jnp.float32),
    ]
    scratch = [
        pltpu.VMEM_SHARED((n_pad, CNT_W), jnp.float32),
        pltpu.VMEM((SUB, CNT_W), jnp.float32),
        pltpu.VMEM((SUB, CNT_W), jnp.float32),
        pltpu.VMEM((GRP, SUB), jnp.int32),
        pltpu.SemaphoreType.DMA,
    ]

    def kern(ud2d, md2d, cnt_m, cnt_u, cacc, ones_v, zc_v, didx_v, ssem):
        tid = lax.axis_index("s")
        cid = lax.axis_index("c")
        ebase = tid * erows_per_tile

        def frow(i, _):
            r = i // (CNT_W // LANES)
            c = lax.rem(i, CNT_W // LANES)
            zc_v[r, pl.ds(c * LANES, LANES)] = jnp.zeros((LANES,), jnp.float32)
            ones_v[r, pl.ds(c * LANES, LANES)] = jnp.ones((LANES,), jnp.float32)
            return 0
        lax.fori_loop(0, SUB * (CNT_W // LANES), frow, 0)
        for q in range(nzc):
            pltpu.sync_copy(zc_v,
                            cacc.at[pl.ds(tid * rows_per_tile + q * SUB, SUB)])
        plsc.subcore_barrier()

        def cnt_loop(d2d):
            def outer(g, _):
                pltpu.sync_copy(d2d.at[pl.ds(ebase + g * GRP, GRP)], didx_v)
                for j in range(GRP):
                    pltpu.async_copy(ones_v, cacc.at[didx_v.at[j]], ssem,
                                     add=True)
                for j in range(GRP):
                    pltpu.make_async_copy(ones_v, cacc.at[pl.ds(0, SUB)],
                                          ssem).wait()
                return 0
            lax.fori_loop(0, erows_per_tile // GRP, outer, 0)

        @pl.when(cid == 0)
        def _():
            cnt_loop(md2d)

        @pl.when(cid == 1)
        def _():
            cnt_loop(ud2d)

        plsc.subcore_barrier()

        def readout(out_ref):
            for q in range(nzc):
                r0 = tid * rows_per_tile + q * SUB
                pltpu.sync_copy(cacc.at[pl.ds(r0, SUB)], out_ref.at[pl.ds(r0, SUB)])

        @pl.when(cid == 0)
        def _():
            readout(cnt_m)

        @pl.when(cid == 1)
        def _():
            readout(cnt_u)

    return pl.kernel(kern, out_type=out_type, mesh=_mesh(),
                     scratch_types=scratch)


def _tc_dense(s_m, cnt_m, x_m, w_l_m, b_m, w_r_m,
              s_u, cnt_u, x_u, w_l_u, b_u, w_r_u, residual):
    n = x_m.shape[0]

    def kern(sm, cm, xm, wlm, bm, wrm, su, cu, xu, wlu, bu, wru, om, ou):
        def one(s_ref, c_ref, x_ref, wl_ref, b_ref, wr_ref, o_ref):
            rc = 1.0 / jnp.clip(c_ref[...][:n, 0:1], 1.0, None)
            mean = s_ref[...][:n] * rc
            y = (jnp.dot(mean, wl_ref[...], preferred_element_type=jnp.float32)
                 + b_ref[...]
                 + jnp.dot(x_ref[...], wr_ref[...],
                           preferred_element_type=jnp.float32))
            if residual:
                y = x_ref[...] + jnp.maximum(y, 0.0)
            o_ref[...] = y
        one(sm, cm, xm, wlm, bm, wrm, om)
        one(su, cu, xu, wlu, bu, wru, ou)

    out = pl.pallas_call(
        kern,
        out_shape=[jax.ShapeDtypeStruct((n, D), jnp.float32),
                   jax.ShapeDtypeStruct((n, D), jnp.float32)],
    )(s_m, cnt_m, x_m, w_l_m, b_m, w_r_m, s_u, cnt_u, x_u, w_l_u, b_u, w_r_u)
    return out[0], out[1]


def kernel(x_user, x_movie, edge_index_rates, edge_index_rev,
           edge_weight_rates, edge_weight_rev,
           W1_um_l, b1_um, W1_um_r, W1_mu_l, b1_mu, W1_mu_r,
           W2_um_l, b2_um, W2_um_r, W2_mu_l, b2_mu, W2_mu_r):
    nu, d = x_user.shape
    nm = x_movie.shape[0]
    e = edge_index_rates.shape[1]
    assert d == D and nu == nm

    n_pad = _ceil_to(nu, NTILES * SUB)
    e_pad = _ceil_to(e, NTILES * SUB * GRP)

    u_idx = edge_index_rates[0].astype(jnp.int32)
    m_idx = edge_index_rates[1].astype(jnp.int32)
    pad = e_pad - e
    if pad:
        fill = jnp.arange(pad, dtype=jnp.int32)
        dummy = nu + fill % (n_pad - nu)
        u_s = jnp.concatenate([u_idx, fill % nu])
        m_s = jnp.concatenate([m_idx, fill % nm])
        u_d = jnp.concatenate([u_idx, dummy])
        m_d = jnp.concatenate([m_idx, dummy])
    else:
        u_s = u_d = u_idx
        m_s = m_d = m_idx

    u_s = u_s.reshape(-1, SUB)
    m_s = m_s.reshape(-1, SUB)
    u_d = u_d.reshape(-1, SUB)
    m_d = m_d.reshape(-1, SUB)

    seg = _sc_segsum(n_pad, e_pad)
    c_m, c_u = _sc_counts(n_pad, e_pad)(u_d, m_d)

    s_m, s_u = seg(u_s, m_s, u_d, m_d, x_user, x_movie)
    res_movie, res_user = _tc_dense(
        s_m, c_m, x_movie, W1_um_l, b1_um.reshape(1, D), W1_um_r,
        s_u, c_u, x_user, W1_mu_l, b1_mu.reshape(1, D), W1_mu_r,
        residual=True)

    s2_m, s2_u = seg(u_s, m_s, u_d, m_d, res_user, res_movie)
    m2, u2 = _tc_dense(
        s2_m, c_m, res_movie, W2_um_l, b2_um.reshape(1, D), W2_um_r,
        s2_u, c_u, res_user, W2_mu_l, b2_mu.reshape(1, D), W2_mu_r,
        residual=False)

    return (u2, m2)


# speedup vs baseline: 8.4397x; 1.0236x over previous
import jax
import jax.numpy as jnp
from jax import lax
from jax.experimental import pallas as pl
from jax.experimental.pallas import tpu as pltpu
from jax.experimental.pallas import tpu_sc as plsc

D = 128
LANES = 16
SUB = 128
NTILES = 16


def _ceil_to(x, m):
    return (x + m - 1) // m * m


def _mesh():
    return plsc.VectorSubcoreMesh(core_axis_name="c", subcore_axis_name="s")


GRP = 32


def _sc_segsum(n_pad, e_pad):
    rows_per_tile = n_pad // NTILES
    nzc = rows_per_tile // SUB
    erows_per_tile = e_pad // (NTILES * SUB)
    npairs = erows_per_tile // 2
    ngrp = erows_per_tile // GRP

    out_type = [
        jax.ShapeDtypeStruct((n_pad, D), jnp.float32),
        jax.ShapeDtypeStruct((n_pad, D), jnp.float32),
    ]
    scratch = [
        pltpu.VMEM_SHARED((n_pad, D), jnp.float32),
        pltpu.VMEM((SUB, D), jnp.float32),
        pltpu.VMEM((SUB, D), jnp.float32),
        pltpu.VMEM((GRP, SUB), jnp.int32),
        pltpu.VMEM((GRP, SUB), jnp.int32),
        pltpu.SemaphoreType.DMA,
        pltpu.SemaphoreType.DMA,
        pltpu.SemaphoreType.DMA,
    ]

    def kern(us2d, ms2d, ud2d, md2d, tab_u, tab_m, out_m, out_u,
             acc, rows_a, rows_b, sidx_v, didx_v, sem_a, sem_b, ssem):
        tid = lax.axis_index("s")
        cid = lax.axis_index("c")
        ebase = tid * erows_per_tile

        def zrow(i, _):
            r = i // (D // LANES)
            c = lax.rem(i, D // LANES)
            rows_a[r, pl.ds(c * LANES, LANES)] = jnp.zeros((LANES,), jnp.float32)
            return 0
        lax.fori_loop(0, SUB * (D // LANES), zrow, 0)
        for q in range(nzc):
            pltpu.sync_copy(rows_a,
                            acc.at[pl.ds(tid * rows_per_tile + q * SUB, SUB)])

        plsc.subcore_barrier()

        def edge_loop(tab, s2d, d2d):
            def load_grp(grp):
                pltpu.sync_copy(s2d.at[pl.ds(ebase + grp * GRP, GRP)], sidx_v)
                pltpu.sync_copy(d2d.at[pl.ds(ebase + grp * GRP, GRP)], didx_v)

            def gather(r, buf, sem):
                return pltpu.async_copy(tab.at[sidx_v.at[r]], buf, sem)

            def drain_g(buf, sem):
                pltpu.make_async_copy(tab.at[pl.ds(0, SUB)], buf, sem).wait()

            def scatter(r, buf):
                pltpu.async_copy(buf, acc.at[didx_v.at[r]], ssem, add=True)

            def drain_s(buf):
                pltpu.make_async_copy(buf, acc.at[pl.ds(0, SUB)], ssem).wait()

            load_grp(0)
            gather(0, rows_a, sem_a)

            def pair(t, _):
                r0 = lax.rem(2 * t, GRP)
                drain_g(rows_a, sem_a)
                gather(r0 + 1, rows_b, sem_b)
                scatter(r0, rows_a)
                drain_g(rows_b, sem_b)
                drain_s(rows_a)

                at_grp_end = lax.rem(t + 1, GRP // 2) == 0

                @pl.when(jnp.logical_not(at_grp_end))
                def _():
                    gather(r0 + 2, rows_a, sem_a)
                    scatter(r0 + 1, rows_b)
                    drain_s(rows_b)

                @pl.when(at_grp_end)
                def _():
                    scatter(r0 + 1, rows_b)
                    drain_s(rows_b)

                    @pl.when(t + 1 < npairs)
                    def _():
                        load_grp((2 * t + 2) // GRP)
                        gather(0, rows_a, sem_a)

                return 0
            lax.fori_loop(0, npairs, pair, 0)

        @pl.when(cid == 0)
        def _():
            edge_loop(tab_u, us2d, md2d)

        @pl.when(cid == 1)
        def _():
            edge_loop(tab_m, ms2d, ud2d)

        plsc.subcore_barrier()

        def readout(out_ref):
            for q in range(nzc):
                r0 = tid * rows_per_tile + q * SUB
                pltpu.sync_copy(acc.at[pl.ds(r0, SUB)], out_ref.at[pl.ds(r0, SUB)])

        @pl.when(cid == 0)
        def _():
            readout(out_m)

        @pl.when(cid == 1)
        def _():
            readout(out_u)

    return pl.kernel(kern, out_type=out_type, mesh=_mesh(),
                     scratch_types=scratch)


CNT_W = 128


def _sc_counts(n_pad, e_pad):
    rows_per_tile = n_pad // NTILES
    nzc = rows_per_tile // SUB
    erows_per_tile = e_pad // (NTILES * SUB)

    out_type = [
        jax.ShapeDtypeStruct((n_pad, CNT_W), jnp.float32),
        jax.ShapeDtypeStruct((n_pad, CNT_W), jnp.float32),
    ]
    scratch = [
        pltpu.VMEM_SHARED((n_pad, CNT_W), jnp.float32),
        pltpu.VMEM((SUB, CNT_W), jnp.float32),
        pltpu.VMEM((SUB, CNT_W), jnp.float32),
        pltpu.VMEM((GRP, SUB), jnp.int32),
        pltpu.SemaphoreType.DMA,
    ]

    def kern(ud2d, md2d, cnt_m, cnt_u, cacc, ones_v, zc_v, didx_v, ssem):
        tid = lax.axis_index("s")
        cid = lax.axis_index("c")
        ebase = tid * erows_per_tile

        def frow(i, _):
            r = i // (CNT_W // LANES)
            c = lax.rem(i, CNT_W // LANES)
            zc_v[r, pl.ds(c * LANES, LANES)] = jnp.zeros((LANES,), jnp.float32)
            ones_v[r, pl.ds(c * LANES, LANES)] = jnp.ones((LANES,), jnp.float32)
            return 0
        lax.fori_loop(0, SUB * (CNT_W // LANES), frow, 0)
        for q in range(nzc):
            pltpu.sync_copy(zc_v,
                            cacc.at[pl.ds(tid * rows_per_tile + q * SUB, SUB)])
        plsc.subcore_barrier()

        def cnt_loop(d2d):
            def outer(g, _):
                pltpu.sync_copy(d2d.at[pl.ds(ebase + g * GRP, GRP)], didx_v)
                for j in range(GRP):
                    pltpu.async_copy(ones_v, cacc.at[didx_v.at[j]], ssem,
                                     add=True)
                for j in range(GRP):
                    pltpu.make_async_copy(ones_v, cacc.at[pl.ds(0, SUB)],
                                          ssem).wait()
                return 0
            lax.fori_loop(0, erows_per_tile // GRP, outer, 0)

        @pl.when(cid == 0)
        def _():
            cnt_loop(md2d)

        @pl.when(cid == 1)
        def _():
            cnt_loop(ud2d)

        plsc.subcore_barrier()

        def readout(out_ref):
            for q in range(nzc):
                r0 = tid * rows_per_tile + q * SUB
                pltpu.sync_copy(cacc.at[pl.ds(r0, SUB)], out_ref.at[pl.ds(r0, SUB)])

        @pl.when(cid == 0)
        def _():
            readout(cnt_m)

        @pl.when(cid == 1)
        def _():
            readout(cnt_u)

    return pl.kernel(kern, out_type=out_type, mesh=_mesh(),
                     scratch_types=scratch)


def _tc_dense(s_m, cnt_m, x_m, w_l_m, b_m, w_r_m,
              s_u, cnt_u, x_u, w_l_u, b_u, w_r_u, residual):
    n = x_m.shape[0]

    def kern(sm, cm, xm, wlm, bm, wrm, su, cu, xu, wlu, bu, wru, om, ou):
        def one(s_ref, c_ref, x_ref, wl_ref, b_ref, wr_ref, o_ref):
            rc = 1.0 / jnp.clip(c_ref[...][:n, 0:1], 1.0, None)
            mean = s_ref[...][:n] * rc
            y = (jnp.dot(mean, wl_ref[...], preferred_element_type=jnp.float32)
                 + b_ref[...]
                 + jnp.dot(x_ref[...], wr_ref[...],
                           preferred_element_type=jnp.float32))
            if residual:
                y = x_ref[...] + jnp.maximum(y, 0.0)
            o_ref[...] = y
        one(sm, cm, xm, wlm, bm, wrm, om)
        one(su, cu, xu, wlu, bu, wru, ou)

    out = pl.pallas_call(
        kern,
        out_shape=[jax.ShapeDtypeStruct((n, D), jnp.float32),
                   jax.ShapeDtypeStruct((n, D), jnp.float32)],
    )(s_m, cnt_m, x_m, w_l_m, b_m, w_r_m, s_u, cnt_u, x_u, w_l_u, b_u, w_r_u)
    return out[0], out[1]


def kernel(x_user, x_movie, edge_index_rates, edge_index_rev,
           edge_weight_rates, edge_weight_rev,
           W1_um_l, b1_um, W1_um_r, W1_mu_l, b1_mu, W1_mu_r,
           W2_um_l, b2_um, W2_um_r, W2_mu_l, b2_mu, W2_mu_r):
    nu, d = x_user.shape
    nm = x_movie.shape[0]
    e = edge_index_rates.shape[1]
    assert d == D and nu == nm

    n_pad = _ceil_to(nu, NTILES * SUB)
    e_pad = _ceil_to(e, NTILES * SUB * GRP)

    u_idx = edge_index_rates[0].astype(jnp.int32)
    m_idx = edge_index_rates[1].astype(jnp.int32)
    pad = e_pad - e
    if pad:
        fill = jnp.arange(pad, dtype=jnp.int32)
        dummy = nu + fill % (n_pad - nu)
        u_s = jnp.concatenate([u_idx, fill % nu])
        m_s = jnp.concatenate([m_idx, fill % nm])
        u_d = jnp.concatenate([u_idx, dummy])
        m_d = jnp.concatenate([m_idx, dummy])
    else:
        u_s = u_d = u_idx
        m_s = m_d = m_idx

    u_s = u_s.reshape(-1, SUB)
    m_s = m_s.reshape(-1, SUB)
    u_d = u_d.reshape(-1, SUB)
    m_d = m_d.reshape(-1, SUB)

    seg = _sc_segsum(n_pad, e_pad)
    c_m, c_u = _sc_counts(n_pad, e_pad)(u_d, m_d)

    s_m, s_u = seg(u_s, m_s, u_d, m_d, x_user, x_movie)
    res_movie, res_user = _tc_dense(
        s_m, c_m, x_movie, W1_um_l, b1_um.reshape(1, D), W1_um_r,
        s_u, c_u, x_user, W1_mu_l, b1_mu.reshape(1, D), W1_mu_r,
        residual=True)

    s2_m, s2_u = seg(u_s, m_s, u_d, m_d, res_user, res_movie)
    m2, u2 = _tc_dense(
        s2_m, c_m, res_movie, W2_um_l, b2_um.reshape(1, D), W2_um_r,
        s2_u, c_u, res_user, W2_mu_l, b2_mu.reshape(1, D), W2_mu_r,
        residual=False)

    return (u2, m2)


# speedup vs baseline: 8.4743x; 1.0041x over previous
import jax
import jax.numpy as jnp
from jax import lax
from jax.experimental import pallas as pl
from jax.experimental.pallas import tpu as pltpu
from jax.experimental.pallas import tpu_sc as plsc

D = 128
LANES = 16
SUB = 128
NTILES = 16


def _ceil_to(x, m):
    return (x + m - 1) // m * m


def _mesh():
    return plsc.VectorSubcoreMesh(core_axis_name="c", subcore_axis_name="s")


GRP = 32


def _sc_segsum(n_pad, e_pad):
    rows_per_tile = n_pad // NTILES
    nzc = rows_per_tile // SUB
    erows_per_tile = e_pad // (NTILES * SUB)
    npairs = erows_per_tile // 2

    out_type = [
        jax.ShapeDtypeStruct((n_pad, D), jnp.float32),
        jax.ShapeDtypeStruct((n_pad, D), jnp.float32),
    ]
    scratch = [
        pltpu.VMEM_SHARED((n_pad, D), jnp.float32),
        pltpu.VMEM((SUB, D), jnp.float32),
        pltpu.VMEM((SUB, D), jnp.float32),
        pltpu.VMEM((GRP, SUB), jnp.int32),
        pltpu.VMEM((GRP, SUB), jnp.int32),
        pltpu.SemaphoreType.DMA,
        pltpu.SemaphoreType.DMA,
        pltpu.SemaphoreType.DMA,
    ]

    def kern(us2d, ms2d, ud2d, md2d, tab_u, tab_m, out_m, out_u,
             acc, rows_a, rows_b, sidx_v, didx_v, sem_a, sem_b, ssem):
        tid = lax.axis_index("s")
        cid = lax.axis_index("c")
        ebase = tid * erows_per_tile

        def zrow(i, _):
            r = i // (D // LANES)
            c = lax.rem(i, D // LANES)
            rows_a[r, pl.ds(c * LANES, LANES)] = jnp.zeros((LANES,), jnp.float32)
            return 0
        lax.fori_loop(0, SUB * (D // LANES), zrow, 0)
        for q in range(nzc):
            pltpu.sync_copy(rows_a,
                            acc.at[pl.ds(tid * rows_per_tile + q * SUB, SUB)])

        plsc.subcore_barrier()

        def edge_loop(tab, s2d, d2d):
            def load_grp(grp):
                pltpu.sync_copy(s2d.at[pl.ds(ebase + grp * GRP, GRP)], sidx_v)
                pltpu.sync_copy(d2d.at[pl.ds(ebase + grp * GRP, GRP)], didx_v)

            def gather(r, buf, sem):
                return pltpu.async_copy(tab.at[sidx_v.at[r]], buf, sem)

            def drain_g(buf, sem):
                pltpu.make_async_copy(tab.at[pl.ds(0, SUB)], buf, sem).wait()

            def scatter(r, buf):
                pltpu.async_copy(buf, acc.at[didx_v.at[r]], ssem, add=True)

            def drain_s(buf):
                pltpu.make_async_copy(buf, acc.at[pl.ds(0, SUB)], ssem).wait()

            load_grp(0)
            gather(0, rows_a, sem_a)

            def pair(t, _):
                r0 = lax.rem(2 * t, GRP)
                drain_g(rows_a, sem_a)
                gather(r0 + 1, rows_b, sem_b)
                scatter(r0, rows_a)
                drain_g(rows_b, sem_b)
                drain_s(rows_a)

                at_grp_end = lax.rem(t + 1, GRP // 2) == 0

                @pl.when(jnp.logical_not(at_grp_end))
                def _():
                    gather(r0 + 2, rows_a, sem_a)
                    scatter(r0 + 1, rows_b)
                    drain_s(rows_b)

                @pl.when(at_grp_end)
                def _():
                    scatter(r0 + 1, rows_b)
                    drain_s(rows_b)

                    @pl.when(t + 1 < npairs)
                    def _():
                        load_grp((2 * t + 2) // GRP)
                        gather(0, rows_a, sem_a)

                return 0
            lax.fori_loop(0, npairs, pair, 0)

        @pl.when(cid == 0)
        def _():
            edge_loop(tab_u, us2d, md2d)

        @pl.when(cid == 1)
        def _():
            edge_loop(tab_m, ms2d, ud2d)

        plsc.subcore_barrier()

        def readout(out_ref):
            for q in range(nzc):
                r0 = tid * rows_per_tile + q * SUB
                pltpu.sync_copy(acc.at[pl.ds(r0, SUB)], out_ref.at[pl.ds(r0, SUB)])

        @pl.when(cid == 0)
        def _():
            readout(out_m)

        @pl.when(cid == 1)
        def _():
            readout(out_u)

    return pl.kernel(kern, out_type=out_type, mesh=_mesh(),
                     scratch_types=scratch)


CNT_W = 128


def _sc_counts(n_pad, e_pad):
    rows_per_tile = n_pad // NTILES
    nzc = rows_per_tile // SUB
    erows_per_tile = e_pad // (NTILES * SUB)

    out_type = [
        jax.ShapeDtypeStruct((n_pad, CNT_W), jnp.float32),
        jax.ShapeDtypeStruct((n_pad, CNT_W), jnp.float32),
    ]
    scratch = [
        pltpu.VMEM_SHARED((n_pad, CNT_W), jnp.float32),
        pltpu.VMEM((SUB, CNT_W), jnp.float32),
        pltpu.VMEM((SUB, CNT_W), jnp.float32),
        pltpu.VMEM((GRP, SUB), jnp.int32),
        pltpu.SemaphoreType.DMA,
    ]

    def kern(ud2d, md2d, cnt_m, cnt_u, cacc, ones_v, zc_v, didx_v, ssem):
        tid = lax.axis_index("s")
        cid = lax.axis_index("c")
        ebase = tid * erows_per_tile

        def frow(i, _):
            r = i // (CNT_W // LANES)
            c = lax.rem(i, CNT_W // LANES)
            zc_v[r, pl.ds(c * LANES, LANES)] = jnp.zeros((LANES,), jnp.float32)
            ones_v[r, pl.ds(c * LANES, LANES)] = jnp.ones((LANES,), jnp.float32)
            return 0
        lax.fori_loop(0, SUB * (CNT_W // LANES), frow, 0)
        for q in range(nzc):
            pltpu.sync_copy(zc_v,
                            cacc.at[pl.ds(tid * rows_per_tile + q * SUB, SUB)])
        plsc.subcore_barrier()

        def cnt_loop(d2d):
            def outer(g, _):
                pltpu.sync_copy(d2d.at[pl.ds(ebase + g * GRP, GRP)], didx_v)
                for j in range(GRP):
                    pltpu.async_copy(ones_v, cacc.at[didx_v.at[j]], ssem,
                                     add=True)
                for j in range(GRP):
                    pltpu.make_async_copy(ones_v, cacc.at[pl.ds(0, SUB)],
                                          ssem).wait()
                return 0
            lax.fori_loop(0, erows_per_tile // GRP, outer, 0)

        @pl.when(cid == 0)
        def _():
            cnt_loop(md2d)

        @pl.when(cid == 1)
        def _():
            cnt_loop(ud2d)

        plsc.subcore_barrier()

        def readout(out_ref):
            for q in range(nzc):
                r0 = tid * rows_per_tile + q * SUB
                pltpu.sync_copy(cacc.at[pl.ds(r0, SUB)], out_ref.at[pl.ds(r0, SUB)])

        @pl.when(cid == 0)
        def _():
            readout(cnt_m)

        @pl.when(cid == 1)
        def _():
            readout(cnt_u)

    return pl.kernel(kern, out_type=out_type, mesh=_mesh(),
                     scratch_types=scratch)


def _tc_dense(s_m, cnt_m, x_m, w_l_m, b_m, w_r_m,
              s_u, cnt_u, x_u, w_l_u, b_u, w_r_u, residual):
    n = x_m.shape[0]

    def kern(sm, cm, xm, wlm, bm, wrm, su, cu, xu, wlu, bu, wru, om, ou):
        def one(s_ref, c_ref, x_ref, wl_ref, b_ref, wr_ref, o_ref):
            rc = 1.0 / jnp.clip(c_ref[...][:n, 0:1], 1.0, None)
            mean = s_ref[...][:n] * rc
            y = (jnp.dot(mean, wl_ref[...], preferred_element_type=jnp.float32)
                 + b_ref[...]
                 + jnp.dot(x_ref[...], wr_ref[...],
                           preferred_element_type=jnp.float32))
            if residual:
                y = x_ref[...] + jnp.maximum(y, 0.0)
            o_ref[...] = y
        one(sm, cm, xm, wlm, bm, wrm, om)
        one(su, cu, xu, wlu, bu, wru, ou)

    out = pl.pallas_call(
        kern,
        out_shape=[jax.ShapeDtypeStruct((n, D), jnp.float32),
                   jax.ShapeDtypeStruct((n, D), jnp.float32)],
    )(s_m, cnt_m, x_m, w_l_m, b_m, w_r_m, s_u, cnt_u, x_u, w_l_u, b_u, w_r_u)
    return out[0], out[1]


def kernel(x_user, x_movie, edge_index_rates, edge_index_rev,
           edge_weight_rates, edge_weight_rev,
           W1_um_l, b1_um, W1_um_r, W1_mu_l, b1_mu, W1_mu_r,
           W2_um_l, b2_um, W2_um_r, W2_mu_l, b2_mu, W2_mu_r):
    nu, d = x_user.shape
    nm = x_movie.shape[0]
    e = edge_index_rates.shape[1]
    assert d == D and nu == nm

    n_pad = _ceil_to(nu, NTILES * SUB)
    e_pad = _ceil_to(e, NTILES * SUB * GRP)

    u_idx = edge_index_rates[0].astype(jnp.int32)
    m_idx = edge_index_rates[1].astype(jnp.int32)
    pad = e_pad - e
    if pad:
        fill = jnp.arange(pad, dtype=jnp.int32)
        dummy = nu + fill % (n_pad - nu)
        u_s = jnp.concatenate([u_idx, fill % nu])
        m_s = jnp.concatenate([m_idx, fill % nm])
        u_d = jnp.concatenate([u_idx, dummy])
        m_d = jnp.concatenate([m_idx, dummy])
    else:
        u_s = u_d = u_idx
        m_s = m_d = m_idx

    u_s = u_s.reshape(-1, SUB)
    m_s = m_s.reshape(-1, SUB)
    u_d = u_d.reshape(-1, SUB)
    m_d = m_d.reshape(-1, SUB)

    seg = _sc_segsum(n_pad, e_pad)
    c_m, c_u = _sc_counts(n_pad, e_pad)(u_d, m_d)

    s_m, s_u = seg(u_s, m_s, u_d, m_d, x_user, x_movie)
    res_movie, res_user = _tc_dense(
        s_m, c_m, x_movie, W1_um_l, b1_um.reshape(1, D), W1_um_r,
        s_u, c_u, x_user, W1_mu_l, b1_mu.reshape(1, D), W1_mu_r,
        residual=True)

    s2_m, s2_u = seg(u_s, m_s, u_d, m_d, res_user, res_movie)
    m2, u2 = _tc_dense(
        s2_m, c_m, res_movie, W2_um_l, b2_um.reshape(1, D), W2_um_r,
        s2_u, c_u, res_user, W2_mu_l, b2_mu.reshape(1, D), W2_mu_r,
        residual=False)

    return (u2, m2)
